# Initial kernel scaffold; baseline (speedup 1.0000x reference)
import jax
import jax.numpy as jnp
from jax.experimental import pallas as pl


def kernel(edge_attr, mask, edge_index, W_edge, b_edge, alpha, bias):
    raise NotImplementedError("write your pallas kernel here")


# speedup vs baseline: 66.8058x; 66.8058x over previous
import functools

import jax
import jax.numpy as jnp
from jax import lax
from jax.experimental import pallas as pl
from jax.experimental.pallas import tpu as pltpu
from jax.experimental.pallas import tpu_sc as plsc

N = 100000
E = 3200000
EDGE_DIM = 16
K = 5

NC, NS, L = 2, 16, 16
NW = NC * NS
NPAD = 100352
EP = 3211264
E_PER_TILE = EP // NW
CHUNK = 2048
N_CHUNKS = E_PER_TILE // CHUNK
NSLICE = NPAD // NS

_mesh = plsc.VectorSubcoreMesh(core_axis_name="c", subcore_axis_name="s")


def _zero_shared_slice(sid, zv, acc):
    zero = jnp.zeros((L,), jnp.float32)

    def zloop(i, carry):
        zv[pl.ds(i * L, L)] = zero
        return carry

    lax.fori_loop(0, NSLICE // L, zloop, 0)
    pltpu.sync_copy(zv, acc.at[pl.ds(sid * NSLICE, NSLICE)])


def _writeback_shared_slice(cid, sid, acc, out_hbm):
    pltpu.sync_copy(
        acc.at[pl.ds(sid * NSLICE, NSLICE)],
        out_hbm.at[pl.ds(cid * NPAD + sid * NSLICE, NSLICE)])


def _deg_body(col_hbm, w_hbm, out_hbm, col_v, w_v, zv, acc):
    cid = lax.axis_index("c")
    sid = lax.axis_index("s")
    wid = cid * NS + sid
    _zero_shared_slice(sid, zv, acc)
    plsc.subcore_barrier()
    base = wid * E_PER_TILE

    def chunk(j, carry):
        e0 = base + j * CHUNK
        pltpu.sync_copy(col_hbm.at[pl.ds(e0, CHUNK)], col_v)
        pltpu.sync_copy(w_hbm.at[pl.ds(e0, CHUNK)], w_v)
        pltpu.sync_copy(w_v, acc.at[col_v], add=True)
        return carry

    lax.fori_loop(0, N_CHUNKS, chunk, 0)
    plsc.subcore_barrier()
    _writeback_shared_slice(cid, sid, acc, out_hbm)


_deg_call = pl.kernel(
    _deg_body,
    out_type=jax.ShapeDtypeStruct((NC * NPAD,), jnp.float32),
    mesh=_mesh,
    scratch_types=[
        pltpu.VMEM((CHUNK,), jnp.int32),
        pltpu.VMEM((CHUNK,), jnp.float32),
        pltpu.VMEM((NSLICE,), jnp.float32),
        pltpu.VMEM_SHARED((NPAD,), jnp.float32),
    ],
)


def _prop_body(y_hbm, row_hbm, col_hbm, w_hbm, out_hbm,
               row_v, col_v, w_v, gath_v, zv, y_sh, acc):
    cid = lax.axis_index("c")
    sid = lax.axis_index("s")
    wid = cid * NS + sid
    pltpu.sync_copy(y_hbm.at[pl.ds(sid * NSLICE, NSLICE)], zv)
    pltpu.sync_copy(zv, y_sh.at[pl.ds(sid * NSLICE, NSLICE)])
    _zero_shared_slice(sid, zv, acc)
    plsc.subcore_barrier()
    base = wid * E_PER_TILE

    def chunk(j, carry):
        e0 = base + j * CHUNK
        pltpu.sync_copy(row_hbm.at[pl.ds(e0, CHUNK)], row_v)
        pltpu.sync_copy(col_hbm.at[pl.ds(e0, CHUNK)], col_v)
        pltpu.sync_copy(w_hbm.at[pl.ds(e0, CHUNK)], w_v)
        pltpu.sync_copy(y_sh.at[row_v], gath_v)

        def mul_group(g, c2):
            s = pl.ds(g * L, L)
            gath_v[s] = gath_v[s] * w_v[s]
            return c2

        lax.fori_loop(0, CHUNK // L, mul_group, 0)
        pltpu.sync_copy(gath_v, acc.at[col_v], add=True)
        return carry

    lax.fori_loop(0, N_CHUNKS, chunk, 0)
    plsc.subcore_barrier()
    _writeback_shared_slice(cid, sid, acc, out_hbm)


_prop_call = pl.kernel(
    _prop_body,
    out_type=jax.ShapeDtypeStruct((NC * NPAD,), jnp.float32),
    mesh=_mesh,
    scratch_types=[
        pltpu.VMEM((CHUNK,), jnp.int32),
        pltpu.VMEM((CHUNK,), jnp.int32),
        pltpu.VMEM((CHUNK,), jnp.float32),
        pltpu.VMEM((CHUNK,), jnp.float32),
        pltpu.VMEM((NSLICE,), jnp.float32),
        pltpu.VMEM_SHARED((NPAD,), jnp.float32),
        pltpu.VMEM_SHARED((NPAD,), jnp.float32),
    ],
)


_EW_BLOCK = 2000
_EW_ROWS = E * EDGE_DIM // 128


def _ew_body(a_ref, sw_ref, b_ref, o_ref):
    acc = jnp.dot(a_ref[...], sw_ref[...],
                  preferred_element_type=jnp.float32,
                  precision=lax.Precision.HIGHEST)
    o_ref[...] = jax.nn.sigmoid(acc + b_ref[0, 0])


_ew_call = pl.pallas_call(
    _ew_body,
    grid=(_EW_ROWS // _EW_BLOCK,),
    in_specs=[
        pl.BlockSpec((_EW_BLOCK, 128), lambda i: (i, 0)),
        pl.BlockSpec((128, 8), lambda i: (0, 0)),
        pl.BlockSpec(memory_space=pltpu.SMEM),
    ],
    out_specs=pl.BlockSpec((_EW_BLOCK, 8), lambda i: (i, 0)),
    out_shape=jax.ShapeDtypeStruct((_EW_ROWS, 8), jnp.float32),
    compiler_params=pltpu.CompilerParams(
        dimension_semantics=("parallel",)),
)


def _prep_body(p0_ref, p1_ref, m_ref, dis_ref, dinv_ref, h_ref, y_ref):
    deg = p0_ref[...] + p1_ref[...] + 1.0
    dinv = 1.0 / deg
    dis = lax.rsqrt(deg)
    h = jnp.maximum(m_ref[...], 0.0)
    dis_ref[...] = dis
    dinv_ref[...] = dinv
    h_ref[...] = h
    y_ref[...] = dis * h


_prep_call = pl.pallas_call(
    _prep_body,
    out_shape=[jax.ShapeDtypeStruct((NPAD // 128, 128), jnp.float32)] * 4,
)


def _update_body(alpha_ref, a0_ref, a1_ref, x_ref, dis_ref, dinv_ref,
                 h_ref, xn_ref, yn_ref):
    alpha = alpha_ref[0]
    agg = dis_ref[...] * (a0_ref[...] + a1_ref[...]) + x_ref[...] * dinv_ref[...]
    xn = agg * (1.0 - alpha) + alpha * h_ref[...]
    xn_ref[...] = xn
    yn_ref[...] = dis_ref[...] * xn


_update_call = pl.pallas_call(
    _update_body,
    in_specs=[
        pl.BlockSpec(memory_space=pltpu.SMEM),
    ] + [pl.BlockSpec((NPAD // 128, 128), lambda: (0, 0))] * 6,
    out_shape=[jax.ShapeDtypeStruct((NPAD // 128, 128), jnp.float32)] * 2,
)


def _finish_body(bias_ref, x_ref, o_ref):
    o_ref[...] = jnp.tanh(x_ref[...] - jax.nn.softplus(bias_ref[0]))


_finish_call = pl.pallas_call(
    _finish_body,
    in_specs=[
        pl.BlockSpec(memory_space=pltpu.SMEM),
        pl.BlockSpec((NPAD // 128, 128), lambda: (0, 0)),
    ],
    out_shape=jax.ShapeDtypeStruct((NPAD // 128, 128), jnp.float32),
)


def kernel(edge_attr, mask, edge_index, W_edge, b_edge, alpha, bias):
    row = edge_index[0].astype(jnp.int32)
    col = edge_index[1].astype(jnp.int32)

    a2d = edge_attr.reshape(_EW_ROWS, 128)
    sw = jnp.kron(jnp.eye(8, dtype=jnp.float32), W_edge.astype(jnp.float32))
    ew = _ew_call(a2d, sw, b_edge.reshape(1, 1)).reshape(E)

    pad = EP - E
    colp = jnp.concatenate([col, jnp.zeros((pad,), jnp.int32)])
    rowp = jnp.concatenate([row, jnp.zeros((pad,), jnp.int32)])
    wp = jnp.concatenate([ew, jnp.zeros((pad,), jnp.float32)])

    degparts = _deg_call(colp, wp)
    p0 = degparts[:NPAD].reshape(NPAD // 128, 128)
    p1 = degparts[NPAD:].reshape(NPAD // 128, 128)
    maskp = jnp.pad(mask.reshape(N), (0, NPAD - N)).reshape(NPAD // 128, 128)

    dis, dinv, h, y = _prep_call(p0, p1, maskp)
    x = h
    alpha1 = alpha.reshape(1).astype(jnp.float32)
    for _ in range(K):
        aggparts = _prop_call(y.reshape(NPAD), rowp, colp, wp)
        a0 = aggparts[:NPAD].reshape(NPAD // 128, 128)
        a1 = aggparts[NPAD:].reshape(NPAD // 128, 128)
        x, y = _update_call(alpha1, a0, a1, x, dis, dinv, h)

    out = _finish_call(bias.reshape(1).astype(jnp.float32), x)
    fill = out.reshape(NPAD)[:N].reshape(N, 1)
    return (fill, ew)


# speedup vs baseline: 84.1899x; 1.2602x over previous
import functools

import jax
import jax.numpy as jnp
from jax import lax
from jax.experimental import pallas as pl
from jax.experimental.pallas import tpu as pltpu
from jax.experimental.pallas import tpu_sc as plsc

N = 100000
E = 3200000
EDGE_DIM = 16
K = 5

NC, NS, L = 2, 16, 16
NW = NC * NS
NPAD = 100352
E_PER_TILE = E // NW
CHUNK = 10000
N_CHUNKS = E_PER_TILE // CHUNK
NSLICE = NPAD // NS

_mesh = plsc.VectorSubcoreMesh(core_axis_name="c", subcore_axis_name="s")


def _zero_shared_slice(sid, zv, acc):
    zero = jnp.zeros((L,), jnp.float32)

    def zloop(i, carry):
        zv[pl.ds(i * L, L)] = zero
        return carry

    lax.fori_loop(0, NSLICE // L, zloop, 0)
    pltpu.sync_copy(zv, acc.at[pl.ds(sid * NSLICE, NSLICE)])


def _writeback_shared_slice(cid, sid, acc, out_hbm):
    pltpu.sync_copy(
        acc.at[pl.ds(sid * NSLICE, NSLICE)],
        out_hbm.at[pl.ds(cid * NPAD + sid * NSLICE, NSLICE)])


def _deg_body(ei_hbm, w_hbm, out_hbm, col_v, w_v, zv, acc, sem):
    cid = lax.axis_index("c")
    sid = lax.axis_index("s")
    wid = cid * NS + sid
    _zero_shared_slice(sid, zv, acc)
    plsc.subcore_barrier()
    base = wid * E_PER_TILE

    def chunk(j, carry):
        e0 = base + j * CHUNK
        c1 = pltpu.async_copy(ei_hbm.at[pl.ds(E + e0, CHUNK)], col_v, sem)
        c2 = pltpu.async_copy(w_hbm.at[pl.ds(e0, CHUNK)], w_v, sem)
        c1.wait()
        c2.wait()
        pltpu.sync_copy(w_v, acc.at[col_v], add=True)
        return carry

    lax.fori_loop(0, N_CHUNKS, chunk, 0)
    plsc.subcore_barrier()
    _writeback_shared_slice(cid, sid, acc, out_hbm)


_deg_call = pl.kernel(
    _deg_body,
    out_type=jax.ShapeDtypeStruct((NC * NPAD,), jnp.float32),
    mesh=_mesh,
    scratch_types=[
        pltpu.VMEM((CHUNK,), jnp.int32),
        pltpu.VMEM((CHUNK,), jnp.float32),
        pltpu.VMEM((NSLICE,), jnp.float32),
        pltpu.VMEM_SHARED((NPAD,), jnp.float32),
        pltpu.SemaphoreType.DMA,
    ],
)


def _prop_body(y_hbm, ei_hbm, w_hbm, out_hbm,
               row_v, col_v, w_v, gath_v, zv, y_sh, acc, sem):
    cid = lax.axis_index("c")
    sid = lax.axis_index("s")
    wid = cid * NS + sid
    pltpu.sync_copy(y_hbm.at[pl.ds(sid * NSLICE, NSLICE)], zv)
    pltpu.sync_copy(zv, y_sh.at[pl.ds(sid * NSLICE, NSLICE)])
    _zero_shared_slice(sid, zv, acc)
    plsc.subcore_barrier()
    base = wid * E_PER_TILE

    def chunk(j, carry):
        e0 = base + j * CHUNK
        c1 = pltpu.async_copy(ei_hbm.at[pl.ds(e0, CHUNK)], row_v, sem)
        c2 = pltpu.async_copy(ei_hbm.at[pl.ds(E + e0, CHUNK)], col_v, sem)
        c3 = pltpu.async_copy(w_hbm.at[pl.ds(e0, CHUNK)], w_v, sem)
        c1.wait()
        pltpu.sync_copy(y_sh.at[row_v], gath_v)
        c3.wait()

        def mul_group(g, c2_):
            s = pl.ds(g * L, L)
            gath_v[s] = gath_v[s] * w_v[s]
            return c2_

        lax.fori_loop(0, CHUNK // L, mul_group, 0)
        c2.wait()
        pltpu.sync_copy(gath_v, acc.at[col_v], add=True)
        return carry

    lax.fori_loop(0, N_CHUNKS, chunk, 0)
    plsc.subcore_barrier()
    _writeback_shared_slice(cid, sid, acc, out_hbm)


_prop_call = pl.kernel(
    _prop_body,
    out_type=jax.ShapeDtypeStruct((NC * NPAD,), jnp.float32),
    mesh=_mesh,
    scratch_types=[
        pltpu.VMEM((CHUNK,), jnp.int32),
        pltpu.VMEM((CHUNK,), jnp.int32),
        pltpu.VMEM((CHUNK,), jnp.float32),
        pltpu.VMEM((CHUNK,), jnp.float32),
        pltpu.VMEM((NSLICE,), jnp.float32),
        pltpu.VMEM_SHARED((NPAD,), jnp.float32),
        pltpu.VMEM_SHARED((NPAD,), jnp.float32),
        pltpu.SemaphoreType.DMA,
    ],
)


_EW_BLOCK = 2000
_EW_ROWS = E * EDGE_DIM // 128


def _ew_body(a_ref, sw_ref, b_ref, o_ref):
    acc = jnp.dot(a_ref[...], sw_ref[...],
                  preferred_element_type=jnp.float32,
                  precision=lax.Precision.HIGHEST)
    o_ref[...] = jax.nn.sigmoid(acc + b_ref[0, 0])


_ew_call = pl.pallas_call(
    _ew_body,
    grid=(_EW_ROWS // _EW_BLOCK,),
    in_specs=[
        pl.BlockSpec((_EW_BLOCK, 128), lambda i: (i, 0)),
        pl.BlockSpec((128, 8), lambda i: (0, 0)),
        pl.BlockSpec(memory_space=pltpu.SMEM),
    ],
    out_specs=pl.BlockSpec((_EW_BLOCK, 8), lambda i: (i, 0)),
    out_shape=jax.ShapeDtypeStruct((_EW_ROWS, 8), jnp.float32),
    compiler_params=pltpu.CompilerParams(
        dimension_semantics=("parallel",)),
)


def _prep_body(p0_ref, p1_ref, m_ref, dis_ref, dinv_ref, h_ref, y_ref):
    deg = p0_ref[...] + p1_ref[...] + 1.0
    dinv = 1.0 / deg
    dis = lax.rsqrt(deg)
    h = jnp.maximum(m_ref[...], 0.0)
    dis_ref[...] = dis
    dinv_ref[...] = dinv
    h_ref[...] = h
    y_ref[...] = dis * h


_prep_call = pl.pallas_call(
    _prep_body,
    out_shape=[jax.ShapeDtypeStruct((NPAD // 128, 128), jnp.float32)] * 4,
)


def _update_body(alpha_ref, a0_ref, a1_ref, x_ref, dis_ref, dinv_ref,
                 h_ref, xn_ref, yn_ref):
    alpha = alpha_ref[0]
    agg = dis_ref[...] * (a0_ref[...] + a1_ref[...]) + x_ref[...] * dinv_ref[...]
    xn = agg * (1.0 - alpha) + alpha * h_ref[...]
    xn_ref[...] = xn
    yn_ref[...] = dis_ref[...] * xn


_update_call = pl.pallas_call(
    _update_body,
    in_specs=[
        pl.BlockSpec(memory_space=pltpu.SMEM),
    ] + [pl.BlockSpec((NPAD // 128, 128), lambda: (0, 0))] * 6,
    out_shape=[jax.ShapeDtypeStruct((NPAD // 128, 128), jnp.float32)] * 2,
)


def _finish_body(bias_ref, x_ref, o_ref):
    o_ref[...] = jnp.tanh(x_ref[...] - jax.nn.softplus(bias_ref[0]))


_finish_call = pl.pallas_call(
    _finish_body,
    in_specs=[
        pl.BlockSpec(memory_space=pltpu.SMEM),
        pl.BlockSpec((NPAD // 128, 128), lambda: (0, 0)),
    ],
    out_shape=jax.ShapeDtypeStruct((NPAD // 128, 128), jnp.float32),
)


def kernel(edge_attr, mask, edge_index, W_edge, b_edge, alpha, bias):
    ei = edge_index.astype(jnp.int32).reshape(2 * E)

    a2d = edge_attr.reshape(_EW_ROWS, 128)
    sw = jnp.kron(jnp.eye(8, dtype=jnp.float32), W_edge.astype(jnp.float32))
    ew = _ew_call(a2d, sw, b_edge.reshape(1, 1)).reshape(E)

    degparts = _deg_call(ei, ew)
    p0 = degparts[:NPAD].reshape(NPAD // 128, 128)
    p1 = degparts[NPAD:].reshape(NPAD // 128, 128)
    maskp = jnp.pad(mask.reshape(N), (0, NPAD - N)).reshape(NPAD // 128, 128)

    dis, dinv, h, y = _prep_call(p0, p1, maskp)
    x = h
    alpha1 = alpha.reshape(1).astype(jnp.float32)
    for _ in range(K):
        aggparts = _prop_call(y.reshape(NPAD), ei, ew)
        a0 = aggparts[:NPAD].reshape(NPAD // 128, 128)
        a1 = aggparts[NPAD:].reshape(NPAD // 128, 128)
        x, y = _update_call(alpha1, a0, a1, x, dis, dinv, h)

    out = _finish_call(bias.reshape(1).astype(jnp.float32), x)
    fill = out.reshape(NPAD)[:N].reshape(N, 1)
    return (fill, ew)
